# Initial kernel scaffold; baseline (speedup 1.0000x reference)
import jax
import jax.numpy as jnp
from jax.experimental import pallas as pl


def kernel(x, embedding):
    raise NotImplementedError("write your pallas kernel here")


# speedup vs baseline: 1.8452x; 1.8452x over previous
import functools

import jax
import jax.numpy as jnp
from jax import lax
from jax.experimental import pallas as pl
from jax.experimental.pallas import tpu as pltpu
from jax.experimental.pallas import tpu_sc as plsc

VOCAB = 1000000
DIM = 64
B_TOTAL = 16384 * 50
NC, NS = 2, 16
NW = NC * NS
B_PER_W = B_TOTAL // NW
CHUNK = 1024
N_CHUNKS = B_PER_W // CHUNK
IDX_W = 128
IDX_ROWS = CHUNK // IDX_W

_mesh = plsc.VectorSubcoreMesh(core_axis_name="c", subcore_axis_name="s")


@functools.partial(
    pl.kernel,
    mesh=_mesh,
    compiler_params=pltpu.CompilerParams(use_tc_tiling_on_sc=False),
    out_type=jax.ShapeDtypeStruct((B_TOTAL, DIM), jnp.float32),
    scratch_types=[
        pltpu.VMEM((IDX_ROWS, IDX_W), jnp.int32),
        pltpu.VMEM((CHUNK, DIM), jnp.float32),
        pltpu.SemaphoreType.DMA,
    ],
)
def _gather(idx_hbm, table_hbm, out_hbm, idx_v, rows_v, sem):
    wid = lax.axis_index("s") * NC + lax.axis_index("c")
    base = wid * B_PER_W

    def body(ci, _):
        off = base + ci * CHUNK
        idx_row = pl.multiple_of(off // IDX_W, 8)
        pltpu.sync_copy(idx_hbm.at[pl.ds(idx_row, IDX_ROWS)], idx_v)
        copies = [
            pltpu.async_copy(
                table_hbm.at[idx_v.at[j]],
                rows_v.at[pl.ds(j * IDX_W, IDX_W)],
                sem,
            )
            for j in range(IDX_ROWS)
        ]
        for c in copies:
            c.wait()
        pltpu.sync_copy(rows_v, out_hbm.at[pl.ds(off, CHUNK)])
        return ()

    lax.fori_loop(0, N_CHUNKS, body, (), unroll=False)


def kernel(x, embedding):
    idx = x.reshape(B_TOTAL // IDX_W, IDX_W).astype(jnp.int32)
    out = _gather(idx, embedding)
    return out.reshape(x.shape[0], x.shape[1], DIM)


# speedup vs baseline: 1.8676x; 1.0122x over previous
import functools

import jax
import jax.numpy as jnp
from jax import lax
from jax.experimental import pallas as pl
from jax.experimental.pallas import tpu as pltpu
from jax.experimental.pallas import tpu_sc as plsc

VOCAB = 1000000
DIM = 64
B_TOTAL = 16384 * 50
NC, NS = 2, 16
NW = NC * NS
B_PER_W = B_TOTAL // NW
IDX_W = 128
IDX_ROWS_W = B_PER_W // IDX_W
CHUNK = 512
K = CHUNK // IDX_W
N_CHUNKS = B_PER_W // CHUNK
NBUF = 2

_mesh = plsc.VectorSubcoreMesh(core_axis_name="c", subcore_axis_name="s")


@functools.partial(
    pl.kernel,
    mesh=_mesh,
    compiler_params=pltpu.CompilerParams(use_tc_tiling_on_sc=False),
    out_type=jax.ShapeDtypeStruct((B_TOTAL, DIM), jnp.float32),
    scratch_types=[
        pltpu.VMEM((IDX_ROWS_W, IDX_W), jnp.int32),
        pltpu.VMEM((NBUF, CHUNK, DIM), jnp.float32),
        pltpu.SemaphoreType.DMA,
        pltpu.SemaphoreType.DMA,
        pltpu.SemaphoreType.DMA,
    ],
)
def _gather(idx_hbm, table_hbm, out_hbm, idx_v, rows_v, sem_g, sem_w0, sem_w1):
    wid = lax.axis_index("s") * NC + lax.axis_index("c")
    base = wid * B_PER_W
    sem_w = (sem_w0, sem_w1)

    idx_row0 = pl.multiple_of(wid * IDX_ROWS_W, 8)
    pltpu.sync_copy(idx_hbm.at[pl.ds(idx_row0, IDX_ROWS_W)], idx_v)

    def step(g, b):
        buf = rows_v.at[b]
        out_off = pl.multiple_of(base + g * CHUNK, CHUNK)

        @pl.when(g >= NBUF)
        def _():
            prev_off = pl.multiple_of(base + (g - NBUF) * CHUNK, CHUNK)
            pltpu.make_async_copy(
                buf, out_hbm.at[pl.ds(prev_off, CHUNK)], sem_w[b]
            ).wait()

        copies = [
            pltpu.async_copy(
                table_hbm.at[idx_v.at[g * K + j]],
                buf.at[pl.ds(j * IDX_W, IDX_W)],
                sem_g,
            )
            for j in range(K)
        ]
        for c in copies:
            c.wait()

        pltpu.make_async_copy(
            buf, out_hbm.at[pl.ds(out_off, CHUNK)], sem_w[b]
        ).start()

    def body(i, _):
        go = i * NBUF
        for b in range(NBUF):
            step(go + b, b)
        return ()

    lax.fori_loop(0, N_CHUNKS // NBUF, body, (), unroll=False)

    for b in range(NBUF):
        g = N_CHUNKS - NBUF + b
        off = pl.multiple_of(base + g * CHUNK, CHUNK)
        pltpu.make_async_copy(
            rows_v.at[b], out_hbm.at[pl.ds(off, CHUNK)], sem_w[b]
        ).wait()


def kernel(x, embedding):
    idx = x.reshape(B_TOTAL // IDX_W, IDX_W).astype(jnp.int32)
    out = _gather(idx, embedding)
    return out.reshape(x.shape[0], x.shape[1], DIM)
